# baseline (speedup 1.0000x reference)
import numpy as np
import jax
import jax.numpy as jnp
from jax.experimental import pallas as pl
from jax.experimental.pallas import tpu as pltpu

ALPHA = 3.0
TOPK = 20
BB, NN, DD = 2, 2048, 512
BR = 256
BV = 512
NBLK = NN // BR


def _rotl32(x, d):
    return ((x << np.uint32(d)) | (x >> np.uint32(32 - d))).astype(np.uint32)


def _threefry2x32(k1, k2, x0, x1):
    ks = [np.uint32(k1), np.uint32(k2), np.uint32(0)]
    ks[2] = np.uint32(ks[0] ^ ks[1] ^ np.uint32(0x1BD11BDA))
    x0 = (x0 + ks[0]).astype(np.uint32)
    x1 = (x1 + ks[1]).astype(np.uint32)
    rotations = [(13, 15, 26, 6), (17, 29, 16, 24)]
    for i in range(5):
        for r in rotations[i % 2]:
            x0 = (x0 + x1).astype(np.uint32)
            x1 = _rotl32(x1, r)
            x1 = (x1 ^ x0).astype(np.uint32)
        x0 = (x0 + ks[(i + 1) % 3]).astype(np.uint32)
        x1 = (x1 + ks[(i + 2) % 3] + np.uint32(i + 1)).astype(np.uint32)
    return x0, x1


def _noise_const():
    n = BB * NN * NN
    flat = np.arange(n, dtype=np.uint64)
    hi = (flat >> np.uint64(32)).astype(np.uint32)
    lo = (flat & np.uint64(0xFFFFFFFF)).astype(np.uint32)
    b0, b1 = _threefry2x32(0, 42, hi, lo)
    bits = b0 ^ b1
    fb = (bits >> np.uint32(9)) | np.uint32(0x3F800000)
    u = np.maximum(np.float32(0.0), fb.view(np.float32) - np.float32(1.0))
    return (u * np.float32(0.01)).reshape(BB, NN, NN)


_NOISE = _noise_const()


def _body(emb_ref, w1_ref, b1_ref, w2_ref, b2_ref, noise_ref, out_ref,
          nv1_s, nv2_s, adj_s, x_s):
    i = pl.program_id(1)
    buf = jax.lax.rem(i, 2)
    pbuf = jax.lax.rem(i + 1, 2)

    @pl.when(i == 0)
    def _compute_nodevecs():
        for c in range(NN // BV):
            e = emb_ref[0, pl.ds(c * BV, BV), :]
            h1 = jax.lax.dot_general(e, w1_ref[...], (((1,), (1,)), ((), ())),
                                     preferred_element_type=jnp.float32)
            nv1_s[pl.ds(c * BV, BV), :] = jnp.tanh(ALPHA * (h1 + b1_ref[...]))
            h2 = jax.lax.dot_general(e, w2_ref[...], (((1,), (1,)), ((), ())),
                                     preferred_element_type=jnp.float32)
            nv2_s[pl.ds(c * BV, BV), :] = jnp.tanh(ALPHA * (h2 + b2_ref[...]))

    rr = jnp.minimum(i, NBLK - 1) * BR
    r1 = nv1_s[pl.ds(rr, BR), :]
    r2 = nv2_s[pl.ds(rr, BR), :]
    s1 = jax.lax.dot_general(r1, nv2_s[...], (((1,), (1,)), ((), ())),
                             preferred_element_type=jnp.float32)
    s2 = jax.lax.dot_general(r2, nv1_s[...], (((1,), (1,)), ((), ())),
                             preferred_element_type=jnp.float32)
    adjp = jnp.maximum(jnp.tanh(ALPHA * (s1 - s2)), 0.0)
    xp = adjp + noise_ref[0]

    adj = adj_s[pbuf]
    x = x_s[pbuf]
    coli = jax.lax.broadcasted_iota(jnp.int32, (BR, NN), 1)
    rcol = (4096 - coli).astype(jnp.float32)
    work = x
    for _ in range(TOPK):
        m = jnp.max(work, axis=1, keepdims=True)
        key = jnp.where(work == m, rcol, 0.0)
        kmax = jnp.max(key, axis=1, keepdims=True)
        work = jnp.where(key == kmax, -1.0, work)
    out_ref[0] = jnp.where(work == x, 0.0, adj)
    adj_s[pl.ds(buf, 1)] = adjp[None]
    x_s[pl.ds(buf, 1)] = xp[None]


def kernel(idx, emb, W1, b1, W2, b2):
    del idx
    clamp_lo = lambda b, i: (b, jnp.maximum(i - 1, 0), 0)
    clamp_hi = lambda b, i: (b, jnp.minimum(i, NBLK - 1), 0)
    out = pl.pallas_call(
        _body,
        grid=(BB, NBLK + 1),
        in_specs=[
            pl.BlockSpec((1, NN, DD), lambda b, i: (b, 0, 0)),
            pl.BlockSpec((DD, DD), lambda b, i: (0, 0)),
            pl.BlockSpec((1, DD), lambda b, i: (0, 0)),
            pl.BlockSpec((DD, DD), lambda b, i: (0, 0)),
            pl.BlockSpec((1, DD), lambda b, i: (0, 0)),
            pl.BlockSpec((1, BR, NN), clamp_hi),
        ],
        out_specs=pl.BlockSpec((1, BR, NN), clamp_lo),
        out_shape=jax.ShapeDtypeStruct((BB, NN, NN), jnp.float32),
        scratch_shapes=[
            pltpu.VMEM((NN, DD), jnp.float32),
            pltpu.VMEM((NN, DD), jnp.float32),
            pltpu.VMEM((2, BR, NN), jnp.float32),
            pltpu.VMEM((2, BR, NN), jnp.float32),
        ],
        interpret=False,
    )(emb, W1, b1.reshape(1, DD), W2, b2.reshape(1, DD), jnp.asarray(_NOISE))
    return out


# speedup vs baseline: 1.1619x; 1.1619x over previous
import numpy as np
import jax
import jax.numpy as jnp
from jax.experimental import pallas as pl
from jax.experimental.pallas import tpu as pltpu

ALPHA = 3.0
TOPK = 20
BB, NN, DD = 2, 2048, 512
BR = 256
BV = 512
NBLK = NN // BR


def _rotl32(x, d):
    return ((x << np.uint32(d)) | (x >> np.uint32(32 - d))).astype(np.uint32)


def _threefry2x32(k1, k2, x0, x1):
    ks = [np.uint32(k1), np.uint32(k2), np.uint32(0)]
    ks[2] = np.uint32(ks[0] ^ ks[1] ^ np.uint32(0x1BD11BDA))
    x0 = (x0 + ks[0]).astype(np.uint32)
    x1 = (x1 + ks[1]).astype(np.uint32)
    rotations = [(13, 15, 26, 6), (17, 29, 16, 24)]
    for i in range(5):
        for r in rotations[i % 2]:
            x0 = (x0 + x1).astype(np.uint32)
            x1 = _rotl32(x1, r)
            x1 = (x1 ^ x0).astype(np.uint32)
        x0 = (x0 + ks[(i + 1) % 3]).astype(np.uint32)
        x1 = (x1 + ks[(i + 2) % 3] + np.uint32(i + 1)).astype(np.uint32)
    return x0, x1


def _noise_const():
    n = BB * NN * NN
    flat = np.arange(n, dtype=np.uint64)
    hi = (flat >> np.uint64(32)).astype(np.uint32)
    lo = (flat & np.uint64(0xFFFFFFFF)).astype(np.uint32)
    b0, b1 = _threefry2x32(0, 42, hi, lo)
    bits = b0 ^ b1
    fb = (bits >> np.uint32(9)) | np.uint32(0x3F800000)
    u = np.maximum(np.float32(0.0), fb.view(np.float32) - np.float32(1.0))
    return (u * np.float32(0.01)).reshape(BB, NN, NN)


_NOISE = _noise_const()


def _body(emb_ref, w1_ref, b1_ref, w2_ref, b2_ref, noise_ref, out_ref,
          nv1_s, nv2_s, adj_s, x_s):
    i = pl.program_id(1)
    buf = jax.lax.rem(i, 2)
    pbuf = jax.lax.rem(i + 1, 2)

    @pl.when(i == 0)
    def _compute_nodevecs():
        for c in range(NN // BV):
            e = emb_ref[0, pl.ds(c * BV, BV), :]
            h1 = jax.lax.dot_general(e, w1_ref[...], (((1,), (1,)), ((), ())),
                                     preferred_element_type=jnp.float32)
            nv1_s[pl.ds(c * BV, BV), :] = jnp.tanh(ALPHA * (h1 + b1_ref[...]))
            h2 = jax.lax.dot_general(e, w2_ref[...], (((1,), (1,)), ((), ())),
                                     preferred_element_type=jnp.float32)
            nv2_s[pl.ds(c * BV, BV), :] = jnp.tanh(ALPHA * (h2 + b2_ref[...]))

    @pl.when(i < NBLK)
    def _score_block():
        r1 = nv1_s[pl.ds(i * BR, BR), :]
        r2 = nv2_s[pl.ds(i * BR, BR), :]
        s1 = jax.lax.dot_general(r1, nv2_s[...], (((1,), (1,)), ((), ())),
                                 preferred_element_type=jnp.float32)
        s2 = jax.lax.dot_general(r2, nv1_s[...], (((1,), (1,)), ((), ())),
                                 preferred_element_type=jnp.float32)
        adj = jnp.maximum(jnp.tanh(ALPHA * (s1 - s2)), 0.0)
        adj_s[pl.ds(buf, 1)] = adj[None]
        x_s[pl.ds(buf, 1)] = (adj + noise_ref[0])[None]

    @pl.when(i > 0)
    def _topk_block():
        adj = adj_s[pbuf]
        x = x_s[pbuf]
        coli = jax.lax.broadcasted_iota(jnp.int32, (BR, NN), 1)
        rcol = (4096 - coli).astype(jnp.float32)
        work = x
        for _ in range(TOPK):
            m = jnp.max(work, axis=1, keepdims=True)
            key = jnp.where(work == m, rcol, 0.0)
            kmax = jnp.max(key, axis=1, keepdims=True)
            work = jnp.where(key == kmax, -1.0, work)
        out_ref[0] = jnp.where(work == x, 0.0, adj)


def kernel(idx, emb, W1, b1, W2, b2):
    del idx
    clamp_lo = lambda b, i: (b, jnp.maximum(i - 1, 0), 0)
    clamp_hi = lambda b, i: (b, jnp.minimum(i, NBLK - 1), 0)
    out = pl.pallas_call(
        _body,
        grid=(BB, NBLK + 1),
        in_specs=[
            pl.BlockSpec((1, NN, DD), lambda b, i: (b, 0, 0)),
            pl.BlockSpec((DD, DD), lambda b, i: (0, 0)),
            pl.BlockSpec((1, DD), lambda b, i: (0, 0)),
            pl.BlockSpec((DD, DD), lambda b, i: (0, 0)),
            pl.BlockSpec((1, DD), lambda b, i: (0, 0)),
            pl.BlockSpec((1, BR, NN), clamp_hi),
        ],
        out_specs=pl.BlockSpec((1, BR, NN), clamp_lo),
        out_shape=jax.ShapeDtypeStruct((BB, NN, NN), jnp.float32),
        scratch_shapes=[
            pltpu.VMEM((NN, DD), jnp.float32),
            pltpu.VMEM((NN, DD), jnp.float32),
            pltpu.VMEM((2, BR, NN), jnp.float32),
            pltpu.VMEM((2, BR, NN), jnp.float32),
        ],
        interpret=False,
    )(emb, W1, b1.reshape(1, DD), W2, b2.reshape(1, DD), jnp.asarray(_NOISE))
    return out


# speedup vs baseline: 1.1663x; 1.0038x over previous
import numpy as np
import jax
import jax.numpy as jnp
from jax.experimental import pallas as pl
from jax.experimental.pallas import tpu as pltpu

ALPHA = 3.0
TOPK = 20
BB, NN, DD = 2, 2048, 512
BR = 256
BV = 512
NBLK = NN // BR


def _rotl32(x, d):
    return ((x << np.uint32(d)) | (x >> np.uint32(32 - d))).astype(np.uint32)


def _threefry2x32(k1, k2, x0, x1):
    ks = [np.uint32(k1), np.uint32(k2), np.uint32(0)]
    ks[2] = np.uint32(ks[0] ^ ks[1] ^ np.uint32(0x1BD11BDA))
    x0 = (x0 + ks[0]).astype(np.uint32)
    x1 = (x1 + ks[1]).astype(np.uint32)
    rotations = [(13, 15, 26, 6), (17, 29, 16, 24)]
    for i in range(5):
        for r in rotations[i % 2]:
            x0 = (x0 + x1).astype(np.uint32)
            x1 = _rotl32(x1, r)
            x1 = (x1 ^ x0).astype(np.uint32)
        x0 = (x0 + ks[(i + 1) % 3]).astype(np.uint32)
        x1 = (x1 + ks[(i + 2) % 3] + np.uint32(i + 1)).astype(np.uint32)
    return x0, x1


def _noise_const():
    n = BB * NN * NN
    flat = np.arange(n, dtype=np.uint64)
    hi = (flat >> np.uint64(32)).astype(np.uint32)
    lo = (flat & np.uint64(0xFFFFFFFF)).astype(np.uint32)
    b0, b1 = _threefry2x32(0, 42, hi, lo)
    bits = b0 ^ b1
    fb = (bits >> np.uint32(9)) | np.uint32(0x3F800000)
    u = np.maximum(np.float32(0.0), fb.view(np.float32) - np.float32(1.0))
    return (u * np.float32(0.01)).reshape(BB, NN, NN)


_NOISE = _noise_const()


def _body(emb_ref, w1_ref, b1_ref, w2_ref, b2_ref, noise_ref, out_ref,
          nv1_s, nv2_s, adj_s, x_s):
    i = pl.program_id(1)
    buf = jax.lax.rem(i, 2)
    pbuf = jax.lax.rem(i + 1, 2)

    @pl.when(i == 0)
    def _compute_nodevecs():
        for c in range(NN // BV):
            e = emb_ref[0, pl.ds(c * BV, BV), :]
            h1 = jax.lax.dot_general(e, w1_ref[...], (((1,), (1,)), ((), ())),
                                     preferred_element_type=jnp.float32)
            nv1_s[pl.ds(c * BV, BV), :] = jnp.tanh(ALPHA * (h1 + b1_ref[...]))
            h2 = jax.lax.dot_general(e, w2_ref[...], (((1,), (1,)), ((), ())),
                                     preferred_element_type=jnp.float32)
            nv2_s[pl.ds(c * BV, BV), :] = jnp.tanh(ALPHA * (h2 + b2_ref[...]))

    @pl.when(i < NBLK)
    def _score_block():
        r1 = nv1_s[pl.ds(i * BR, BR), :]
        r2 = nv2_s[pl.ds(i * BR, BR), :]
        s1 = jax.lax.dot_general(r1, nv2_s[...], (((1,), (1,)), ((), ())),
                                 preferred_element_type=jnp.float32)
        s2 = jax.lax.dot_general(r2, nv1_s[...], (((1,), (1,)), ((), ())),
                                 preferred_element_type=jnp.float32)
        adj = jnp.maximum(jnp.tanh(ALPHA * (s1 - s2)), 0.0)
        adj_s[pl.ds(buf, 1)] = adj[None]
        x_s[pl.ds(buf, 1)] = (adj + noise_ref[0])[None]

    @pl.when(i > 0)
    def _topk_block():
        adj = adj_s[pbuf]
        x = x_s[pbuf]
        coli = jax.lax.broadcasted_iota(jnp.int32, (BR, NN), 1)
        rcol = (4096 - coli).astype(jnp.float32)
        work = x
        for _ in range(TOPK):
            m = jnp.max(work, axis=1, keepdims=True)
            key = jnp.where(work == m, rcol, 0.0)
            kmax = jnp.max(key, axis=1, keepdims=True)
            work = jnp.where(key == kmax, -1.0, work)
        out_ref[0] = jnp.where(work < 0.0, adj, 0.0)


def kernel(idx, emb, W1, b1, W2, b2):
    del idx
    clamp_lo = lambda b, i: (b, jnp.maximum(i - 1, 0), 0)
    clamp_hi = lambda b, i: (b, jnp.minimum(i, NBLK - 1), 0)
    out = pl.pallas_call(
        _body,
        grid=(BB, NBLK + 1),
        in_specs=[
            pl.BlockSpec((1, NN, DD), lambda b, i: (b, 0, 0)),
            pl.BlockSpec((DD, DD), lambda b, i: (0, 0)),
            pl.BlockSpec((1, DD), lambda b, i: (0, 0)),
            pl.BlockSpec((DD, DD), lambda b, i: (0, 0)),
            pl.BlockSpec((1, DD), lambda b, i: (0, 0)),
            pl.BlockSpec((1, BR, NN), clamp_hi),
        ],
        out_specs=pl.BlockSpec((1, BR, NN), clamp_lo),
        out_shape=jax.ShapeDtypeStruct((BB, NN, NN), jnp.float32),
        scratch_shapes=[
            pltpu.VMEM((NN, DD), jnp.float32),
            pltpu.VMEM((NN, DD), jnp.float32),
            pltpu.VMEM((2, BR, NN), jnp.float32),
            pltpu.VMEM((2, BR, NN), jnp.float32),
        ],
        interpret=False,
    )(emb, W1, b1.reshape(1, DD), W2, b2.reshape(1, DD), jnp.asarray(_NOISE))
    return out
